# baseline (speedup 1.0000x reference)
import functools

import jax
import jax.numpy as jnp
from jax.experimental import pallas as pl

N = 10000
E = 320000
F = 128
H = 256
D = 128


def _decoder_mm_body(nx_a, nx_b, o_ref):
    a = nx_a[...]
    b = nx_b[...]
    o_ref[...] = jax.lax.dot_general(
        a, b, (((1,), (1,)), ((), ())), preferred_element_type=jnp.float32)


def _decoder_matmul(node_x):
    TM = 1024
    grid = (pl.cdiv(N, TM), pl.cdiv(N, TM))
    return pl.pallas_call(
        _decoder_mm_body,
        grid=grid,
        in_specs=[
            pl.BlockSpec((TM, H), lambda i, j: (i, 0)),
            pl.BlockSpec((TM, H), lambda i, j: (j, 0)),
        ],
        out_specs=pl.BlockSpec((TM, TM), lambda i, j: (i, j)),
        out_shape=jax.ShapeDtypeStruct((N, N), jnp.float32),
    )(node_x, node_x)


def _gcn_conv(x, src, dst, W, b, n):
    xw = x @ W
    loop = jnp.arange(n)
    s = jnp.concatenate([src, loop])
    d = jnp.concatenate([dst, loop])
    deg = jax.ops.segment_sum(jnp.ones(s.shape[0], x.dtype), d, num_segments=n)
    dis = jnp.where(deg > 0, 1.0 / jnp.sqrt(deg), 0.0)
    norm = dis[s] * dis[d]
    out = jax.ops.segment_sum(xw[s] * norm[:, None], d, num_segments=n)
    return out + b


def _gatv2_conv(x, src, dst, edge_attr, Wl, bl, Wr, br, We, att, b, n):
    xl = x @ Wl + bl
    xr = x @ Wr + br
    loop = jnp.arange(n)
    s = jnp.concatenate([src, loop])
    d = jnp.concatenate([dst, loop])
    loop_attr = jnp.broadcast_to(edge_attr.mean(axis=0, keepdims=True), (n, edge_attr.shape[1]))
    ea = jnp.concatenate([edge_attr, loop_attr], axis=0)
    ef = ea @ We
    m = xl[s] + xr[d] + ef
    m = jax.nn.leaky_relu(m, 0.2)
    logit = m @ att
    mx = jax.ops.segment_max(logit, d, num_segments=n)
    al = jnp.exp(logit - mx[d])
    den = jax.ops.segment_sum(al, d, num_segments=n)
    alpha = al / den[d]
    out = jax.ops.segment_sum(xl[s] * alpha[:, None], d, num_segments=n)
    return out + b


def kernel(x, edge_index, edge_attr, batch, gcn_W, gcn_b, gat_Wl, gat_bl,
           gat_Wr, gat_br, gat_We, gat_att, gat_b, gcn2_W, gcn2_b, lin1_W, lin1_b):
    src = edge_index[0]
    dst = edge_index[1]
    n = x.shape[0]
    h = jax.nn.relu(_gcn_conv(x, src, dst, gcn_W, gcn_b, n))
    h = jax.nn.relu(_gatv2_conv(h, src, dst, edge_attr, gat_Wl, gat_bl,
                                gat_Wr, gat_br, gat_We, gat_att, gat_b, n))
    z = _gcn_conv(h, src, dst, gcn2_W, gcn2_b, n)
    node_x = jax.nn.relu(z @ lin1_W + lin1_b)
    return _decoder_matmul(node_x)


# speedup vs baseline: 2.0142x; 2.0142x over previous
import functools

import jax
import jax.numpy as jnp
from jax import lax
from jax.experimental import pallas as pl
from jax.experimental.pallas import tpu as pltpu
from jax.experimental.pallas import tpu_sc as plsc

N = 10000
E = 320000
F = 128
H = 256
D = 128

NC = 2
NS = 16
L = 16
CH = 200

RPT_A = 632
RPT_LAST = N - (NS - 1) * RPT_A

_SC_MESH = plsc.VectorSubcoreMesh(core_axis_name="c", subcore_axis_name="s")


def _tile_rows(s, fn):
    @pl.when(s < NS - 1)
    def _():
        fn(pl.multiple_of(s * RPT_A, 8), RPT_A)

    @pl.when(s == NS - 1)
    def _():
        fn((NS - 1) * RPT_A, RPT_LAST)


@functools.partial(
    pl.kernel,
    out_type=(jax.ShapeDtypeStruct((N,), jnp.float32),
              jax.ShapeDtypeStruct((N,), jnp.float32)),
    mesh=_SC_MESH,
    scratch_types=[
        pltpu.VMEM((CH,), jnp.int32),
        pltpu.VMEM((CH,), jnp.float32),
        pltpu.VMEM_SHARED((N,), jnp.float32),
    ],
)
def _deg_kernel(dst_hbm, zeros_hbm, out0_hbm, out1_hbm, idx_v, ones_v, acc):
    c = lax.axis_index("c")
    s = lax.axis_index("s")
    ept = E // (NC * NS)

    @pl.when(s == 0)
    def _():
        pltpu.sync_copy(zeros_hbm, acc)

    for j in range(CH // L):
        ones_v[pl.ds(j * L, L)] = jnp.ones((L,), jnp.float32)
    plsc.subcore_barrier()

    base = (c * NS + s) * ept

    def chunk(g, _):
        pltpu.sync_copy(dst_hbm.at[pl.ds(base + g * CH, CH)], idx_v)
        pltpu.sync_copy(ones_v, acc.at[idx_v], add=True)
        return ()

    lax.fori_loop(0, ept // CH, chunk, ())
    plsc.subcore_barrier()

    @pl.when((s == 0) & (c == 0))
    def _():
        pltpu.sync_copy(acc, out0_hbm)

    @pl.when((s == 0) & (c == 1))
    def _():
        pltpu.sync_copy(acc, out1_hbm)


_EPT_COL = E // NS


@functools.partial(
    pl.kernel,
    out_type=jax.ShapeDtypeStruct((N, 256), jnp.float32),
    mesh=_SC_MESH,
    scratch_types=[
        pltpu.VMEM((CH,), jnp.int32),
        pltpu.VMEM((CH,), jnp.int32),
        pltpu.VMEM((CH, 128), jnp.float32),
        pltpu.VMEM_SHARED((N, 128), jnp.float32),
        pltpu.SemaphoreType.DMA,
    ],
)
def _colsplit_scatter(table_hbm, src_hbm, dst_hbm, zeros_hbm, out_hbm,
                      idx_s, idx_d, rows, acc, sem):
    c = lax.axis_index("c")
    s = lax.axis_index("s")

    def zero(off, sz):
        pltpu.sync_copy(zeros_hbm.at[pl.ds(off, sz)], acc.at[pl.ds(off, sz)])

    _tile_rows(s, zero)
    plsc.subcore_barrier()
    base = s * _EPT_COL

    def chunk(g, _):
        off = base + g * CH
        pltpu.sync_copy(src_hbm.at[pl.ds(off, CH)], idx_s)

        def adj(j, _):
            sl = pl.ds(j * L, L)
            idx_s[sl] = idx_s[sl] + c * N
            return ()

        lax.fori_loop(0, CH // L, adj, ())
        pltpu.sync_copy(dst_hbm.at[pl.ds(off, CH)], idx_d)
        pltpu.async_copy(table_hbm.at[idx_s], rows, sem).wait()
        pltpu.sync_copy(rows, acc.at[idx_d], add=True)
        return ()

    lax.fori_loop(0, _EPT_COL // CH, chunk, ())
    plsc.subcore_barrier()

    def wb(off, sz):
        pltpu.sync_copy(
            acc.at[pl.ds(off, sz)],
            out_hbm.at[pl.ds(off, sz), pl.ds(pl.multiple_of(c * 128, 128), 128)])

    _tile_rows(s, wb)


_EPT_EDGE = E // (NC * NS)


@functools.partial(
    pl.kernel,
    out_type=(jax.ShapeDtypeStruct((N, 128), jnp.float32),
              jax.ShapeDtypeStruct((N, 128), jnp.float32)),
    mesh=_SC_MESH,
    scratch_types=[
        pltpu.VMEM((CH,), jnp.int32),
        pltpu.VMEM((CH,), jnp.int32),
        pltpu.VMEM((CH, 128), jnp.float32),
        pltpu.VMEM_SHARED((N, 128), jnp.float32),
        pltpu.SemaphoreType.DMA,
    ],
)
def _edgesplit_scatter(table_hbm, src_hbm, dst_hbm, zeros_hbm,
                       out0_hbm, out1_hbm, idx_s, idx_d, rows, acc, sem):
    c = lax.axis_index("c")
    s = lax.axis_index("s")

    def zero(off, sz):
        pltpu.sync_copy(zeros_hbm.at[pl.ds(off, sz)], acc.at[pl.ds(off, sz)])

    _tile_rows(s, zero)
    plsc.subcore_barrier()
    base = (c * NS + s) * _EPT_EDGE

    def chunk(g, _):
        off = base + g * CH
        pltpu.sync_copy(src_hbm.at[pl.ds(off, CH)], idx_s)
        pltpu.sync_copy(dst_hbm.at[pl.ds(off, CH)], idx_d)
        pltpu.async_copy(table_hbm.at[idx_s], rows, sem).wait()
        pltpu.sync_copy(rows, acc.at[idx_d], add=True)
        return ()

    lax.fori_loop(0, _EPT_EDGE // CH, chunk, ())
    plsc.subcore_barrier()

    def wb0(off, sz):
        pltpu.sync_copy(acc.at[pl.ds(off, sz)], out0_hbm.at[pl.ds(off, sz)])

    def wb1(off, sz):
        pltpu.sync_copy(acc.at[pl.ds(off, sz)], out1_hbm.at[pl.ds(off, sz)])

    @pl.when(c == 0)
    def _():
        _tile_rows(s, wb0)

    @pl.when(c == 1)
    def _():
        _tile_rows(s, wb1)


def _sc_gcn_scatter(xw_scaled, src, dst):
    w = xw_scaled.shape[1]
    zeros = jnp.zeros((N, 128), jnp.float32)
    if w == 256:
        table2 = jnp.concatenate([xw_scaled[:, :128], xw_scaled[:, 128:]], axis=0)
        return _colsplit_scatter(table2, src, dst, zeros)
    assert w == 128
    p0, p1 = _edgesplit_scatter(xw_scaled, src, dst, zeros)
    return p0 + p1


def _decoder_mm_body(nx_a, nx_b, o_ref):
    a = nx_a[...]
    b = nx_b[...]
    o_ref[...] = jax.lax.dot_general(
        a, b, (((1,), (1,)), ((), ())), preferred_element_type=jnp.float32)


def _decoder_matmul(node_x):
    TM = 1024
    grid = (pl.cdiv(N, TM), pl.cdiv(N, TM))
    return pl.pallas_call(
        _decoder_mm_body,
        grid=grid,
        in_specs=[
            pl.BlockSpec((TM, H), lambda i, j: (i, 0)),
            pl.BlockSpec((TM, H), lambda i, j: (j, 0)),
        ],
        out_specs=pl.BlockSpec((TM, TM), lambda i, j: (i, j)),
        out_shape=jax.ShapeDtypeStruct((N, N), jnp.float32),
    )(node_x, node_x)


def _gcn_conv_sc(x, src, dst, W, b, dis):
    xw = x @ W
    xwp = xw * dis[:, None]
    scat = _sc_gcn_scatter(xwp, src, dst)
    return dis[:, None] * (scat + xwp) + b


def _gatv2_conv(x, src, dst, edge_attr, Wl, bl, Wr, br, We, att, b, n):
    xl = x @ Wl + bl
    xr = x @ Wr + br
    loop = jnp.arange(n)
    s = jnp.concatenate([src, loop])
    d = jnp.concatenate([dst, loop])
    loop_attr = jnp.broadcast_to(edge_attr.mean(axis=0, keepdims=True), (n, edge_attr.shape[1]))
    ea = jnp.concatenate([edge_attr, loop_attr], axis=0)
    ef = ea @ We
    m = xl[s] + xr[d] + ef
    m = jax.nn.leaky_relu(m, 0.2)
    logit = m @ att
    mx = jax.ops.segment_max(logit, d, num_segments=n)
    al = jnp.exp(logit - mx[d])
    den = jax.ops.segment_sum(al, d, num_segments=n)
    alpha = al / den[d]
    out = jax.ops.segment_sum(xl[s] * alpha[:, None], d, num_segments=n)
    return out + b


def kernel(x, edge_index, edge_attr, batch, gcn_W, gcn_b, gat_Wl, gat_bl,
           gat_Wr, gat_br, gat_We, gat_att, gat_b, gcn2_W, gcn2_b, lin1_W, lin1_b):
    src = edge_index[0]
    dst = edge_index[1]
    n = x.shape[0]

    deg_part = _deg_kernel(dst, jnp.zeros((N,), jnp.float32))
    deg = deg_part[0] + deg_part[1] + 1.0
    dis = lax.rsqrt(deg)

    h = jax.nn.relu(_gcn_conv_sc(x, src, dst, gcn_W, gcn_b, dis))
    h = jax.nn.relu(_gatv2_conv(h, src, dst, edge_attr, gat_Wl, gat_bl,
                                gat_Wr, gat_br, gat_We, gat_att, gat_b, n))
    z = _gcn_conv_sc(h, src, dst, gcn2_W, gcn2_b, dis)
    node_x = jax.nn.relu(z @ lin1_W + lin1_b)
    return _decoder_matmul(node_x)


# speedup vs baseline: 6.8296x; 3.3908x over previous
import functools

import jax
import jax.numpy as jnp
from jax import lax
from jax.experimental import pallas as pl
from jax.experimental.pallas import tpu as pltpu
from jax.experimental.pallas import tpu_sc as plsc

N = 10000
E = 320000
F = 128
H = 256
D = 128

NC = 2
NS = 16
L = 16
CH = 200
CHS = 400

RPT_A = 632
RPT_LAST = N - (NS - 1) * RPT_A

_SC_MESH = plsc.VectorSubcoreMesh(core_axis_name="c", subcore_axis_name="s")


def _tile_rows(s, fn):
    @pl.when(s < NS - 1)
    def _():
        fn(pl.multiple_of(s * RPT_A, 8), RPT_A)

    @pl.when(s == NS - 1)
    def _():
        fn((NS - 1) * RPT_A, RPT_LAST)


@functools.partial(
    pl.kernel,
    out_type=(jax.ShapeDtypeStruct((N,), jnp.float32),
              jax.ShapeDtypeStruct((N,), jnp.float32)),
    mesh=_SC_MESH,
    scratch_types=[
        pltpu.VMEM((CHS,), jnp.int32),
        pltpu.VMEM((CHS,), jnp.float32),
        pltpu.VMEM_SHARED((N,), jnp.float32),
    ],
)
def _deg_kernel(dst_hbm, zeros_hbm, out0_hbm, out1_hbm, idx_v, ones_v, acc):
    c = lax.axis_index("c")
    s = lax.axis_index("s")
    ept = E // (NC * NS)

    @pl.when(s == 0)
    def _():
        pltpu.sync_copy(zeros_hbm, acc)

    for j in range(CHS // L):
        ones_v[pl.ds(j * L, L)] = jnp.ones((L,), jnp.float32)
    plsc.subcore_barrier()

    base = (c * NS + s) * ept

    def chunk(g, _):
        pltpu.sync_copy(dst_hbm.at[pl.ds(base + g * CHS, CHS)], idx_v)
        pltpu.sync_copy(ones_v, acc.at[idx_v], add=True)
        return ()

    lax.fori_loop(0, ept // CHS, chunk, ())
    plsc.subcore_barrier()

    @pl.when((s == 0) & (c == 0))
    def _():
        pltpu.sync_copy(acc, out0_hbm)

    @pl.when((s == 0) & (c == 1))
    def _():
        pltpu.sync_copy(acc, out1_hbm)


_EPT_COL = E // NS


@functools.partial(
    pl.kernel,
    out_type=jax.ShapeDtypeStruct((N, 256), jnp.float32),
    mesh=_SC_MESH,
    scratch_types=[
        pltpu.VMEM((CH,), jnp.int32),
        pltpu.VMEM((CH,), jnp.int32),
        pltpu.VMEM((CH, 128), jnp.float32),
        pltpu.VMEM_SHARED((N, 128), jnp.float32),
        pltpu.SemaphoreType.DMA,
    ],
)
def _colsplit_scatter(table_hbm, src_hbm, dst_hbm, zeros_hbm, out_hbm,
                      idx_s, idx_d, rows, acc, sem):
    c = lax.axis_index("c")
    s = lax.axis_index("s")

    def zero(off, sz):
        pltpu.sync_copy(zeros_hbm.at[pl.ds(off, sz)], acc.at[pl.ds(off, sz)])

    _tile_rows(s, zero)
    plsc.subcore_barrier()
    base = s * _EPT_COL

    def chunk(g, _):
        off = base + g * CH
        pltpu.sync_copy(src_hbm.at[pl.ds(c * E + off, CH)], idx_s)
        pltpu.sync_copy(dst_hbm.at[pl.ds(off, CH)], idx_d)
        pltpu.async_copy(table_hbm.at[idx_s], rows, sem).wait()
        pltpu.sync_copy(rows, acc.at[idx_d], add=True)
        return ()

    lax.fori_loop(0, _EPT_COL // CH, chunk, ())
    plsc.subcore_barrier()

    def wb(off, sz):
        pltpu.sync_copy(
            acc.at[pl.ds(off, sz)],
            out_hbm.at[pl.ds(off, sz), pl.ds(pl.multiple_of(c * 128, 128), 128)])

    _tile_rows(s, wb)


_EPT_EDGE = E // (NC * NS)


@functools.partial(
    pl.kernel,
    out_type=(jax.ShapeDtypeStruct((N, 128), jnp.float32),
              jax.ShapeDtypeStruct((N, 128), jnp.float32)),
    mesh=_SC_MESH,
    scratch_types=[
        pltpu.VMEM((CH,), jnp.int32),
        pltpu.VMEM((CH,), jnp.int32),
        pltpu.VMEM((CH, 128), jnp.float32),
        pltpu.VMEM_SHARED((N, 128), jnp.float32),
        pltpu.SemaphoreType.DMA,
    ],
)
def _edgesplit_scatter(table_hbm, src_hbm, dst_hbm, zeros_hbm,
                       out0_hbm, out1_hbm, idx_s, idx_d, rows, acc, sem):
    c = lax.axis_index("c")
    s = lax.axis_index("s")

    def zero(off, sz):
        pltpu.sync_copy(zeros_hbm.at[pl.ds(off, sz)], acc.at[pl.ds(off, sz)])

    _tile_rows(s, zero)
    plsc.subcore_barrier()
    base = (c * NS + s) * _EPT_EDGE

    def chunk(g, _):
        off = base + g * CH
        pltpu.sync_copy(src_hbm.at[pl.ds(off, CH)], idx_s)
        pltpu.sync_copy(dst_hbm.at[pl.ds(off, CH)], idx_d)
        pltpu.async_copy(table_hbm.at[idx_s], rows, sem).wait()
        pltpu.sync_copy(rows, acc.at[idx_d], add=True)
        return ()

    lax.fori_loop(0, _EPT_EDGE // CH, chunk, ())
    plsc.subcore_barrier()

    def wb0(off, sz):
        pltpu.sync_copy(acc.at[pl.ds(off, sz)], out0_hbm.at[pl.ds(off, sz)])

    def wb1(off, sz):
        pltpu.sync_copy(acc.at[pl.ds(off, sz)], out1_hbm.at[pl.ds(off, sz)])

    @pl.when(c == 0)
    def _():
        _tile_rows(s, wb0)

    @pl.when(c == 1)
    def _():
        _tile_rows(s, wb1)


@functools.partial(
    pl.kernel,
    out_type=jax.ShapeDtypeStruct((NC, E, 128), jnp.float32),
    mesh=_SC_MESH,
    scratch_types=[
        pltpu.VMEM((CH,), jnp.int32),
        pltpu.VMEM((CH,), jnp.int32),
        pltpu.VMEM((CH, 128), jnp.float32),
        pltpu.VMEM((CH, 128), jnp.float32),
        pltpu.SemaphoreType.DMA,
        pltpu.SemaphoreType.DMA,
    ],
)
def _gat_pass1(xl2_hbm, xr2_hbm, src_hbm, dst_hbm, g_hbm,
               idx_s, idx_d, rows, rows2, sem, sem2):
    c = lax.axis_index("c")
    s = lax.axis_index("s")
    base = s * _EPT_COL

    def chunk(g, _):
        off = base + g * CH
        pltpu.sync_copy(src_hbm.at[pl.ds(c * E + off, CH)], idx_s)
        pltpu.sync_copy(dst_hbm.at[pl.ds(c * E + off, CH)], idx_d)
        cp1 = pltpu.async_copy(xl2_hbm.at[idx_s], rows, sem)
        cp2 = pltpu.async_copy(xr2_hbm.at[idx_d], rows2, sem2)
        cp1.wait()
        cp2.wait()

        def vadd(e, _):
            for k in range(128 // L):
                sl = pl.ds(k * L, L)
                rows[e, sl] = rows[e, sl] + rows2[e, sl]
            return ()

        lax.fori_loop(0, CH, vadd, ())
        pltpu.sync_copy(rows, g_hbm.at[c, pl.ds(off, CH)])
        return ()

    lax.fori_loop(0, _EPT_COL // CH, chunk, ())


@functools.partial(
    pl.kernel,
    out_type=(jax.ShapeDtypeStruct((N,), jnp.float32),
              jax.ShapeDtypeStruct((N,), jnp.float32)),
    mesh=_SC_MESH,
    scratch_types=[
        pltpu.VMEM((CH,), jnp.int32),
        pltpu.VMEM((CH,), jnp.float32),
        pltpu.VMEM_SHARED((N,), jnp.float32),
    ],
)
def _val_segsum(vals_hbm, dst_hbm, zeros_hbm, out0_hbm, out1_hbm,
                idx_v, val_v, acc):
    c = lax.axis_index("c")
    s = lax.axis_index("s")

    @pl.when(s == 0)
    def _():
        pltpu.sync_copy(zeros_hbm, acc)

    plsc.subcore_barrier()
    base = (c * NS + s) * _EPT_EDGE

    def chunk(g, _):
        off = base + g * CH
        pltpu.sync_copy(dst_hbm.at[pl.ds(off, CH)], idx_v)
        pltpu.sync_copy(vals_hbm.at[pl.ds(off, CH)], val_v)
        pltpu.sync_copy(val_v, acc.at[idx_v], add=True)
        return ()

    lax.fori_loop(0, _EPT_EDGE // CH, chunk, ())
    plsc.subcore_barrier()

    @pl.when((s == 0) & (c == 0))
    def _():
        pltpu.sync_copy(acc, out0_hbm)

    @pl.when((s == 0) & (c == 1))
    def _():
        pltpu.sync_copy(acc, out1_hbm)


@functools.partial(
    pl.kernel,
    out_type=jax.ShapeDtypeStruct((N, 256), jnp.float32),
    mesh=_SC_MESH,
    scratch_types=[
        pltpu.VMEM((CH,), jnp.int32),
        pltpu.VMEM((CH,), jnp.int32),
        pltpu.VMEM((CH * L,), jnp.float32),
        pltpu.VMEM((CH, 128), jnp.float32),
        pltpu.VMEM_SHARED((N, 128), jnp.float32),
        pltpu.SemaphoreType.DMA,
    ],
)
def _colsplit_scatter_w(table_hbm, src_hbm, dst_hbm, w16_hbm, zeros_hbm, out_hbm,
                        idx_s, idx_d, w_v, rows, acc, sem):
    c = lax.axis_index("c")
    s = lax.axis_index("s")

    def zero(off, sz):
        pltpu.sync_copy(zeros_hbm.at[pl.ds(off, sz)], acc.at[pl.ds(off, sz)])

    _tile_rows(s, zero)
    plsc.subcore_barrier()
    base = s * _EPT_COL

    def chunk(g, _):
        off = base + g * CH
        pltpu.sync_copy(src_hbm.at[pl.ds(c * E + off, CH)], idx_s)
        pltpu.sync_copy(dst_hbm.at[pl.ds(off, CH)], idx_d)
        pltpu.sync_copy(w16_hbm.at[pl.ds(off * L, CH * L)], w_v)
        pltpu.async_copy(table_hbm.at[idx_s], rows, sem).wait()

        def scale(e, _):
            wv = w_v[pl.ds(e * L, L)]
            for k in range(128 // L):
                sl = pl.ds(k * L, L)
                rows[e, sl] = rows[e, sl] * wv
            return ()

        lax.fori_loop(0, CH, scale, ())
        pltpu.sync_copy(rows, acc.at[idx_d], add=True)
        return ()

    lax.fori_loop(0, _EPT_COL // CH, chunk, ())
    plsc.subcore_barrier()

    def wb(off, sz):
        pltpu.sync_copy(
            acc.at[pl.ds(off, sz)],
            out_hbm.at[pl.ds(off, sz), pl.ds(pl.multiple_of(c * 128, 128), 128)])

    _tile_rows(s, wb)


def _sc_gcn_scatter(xw_scaled, src2, dst):
    w = xw_scaled.shape[1]
    zeros = jnp.zeros((N, 128), jnp.float32)
    if w == 256:
        table2 = jnp.concatenate([xw_scaled[:, :128], xw_scaled[:, 128:]], axis=0)
        return _colsplit_scatter(table2, src2, dst, zeros)
    assert w == 128
    p0, p1 = _edgesplit_scatter(xw_scaled, src2[:E], dst, zeros)
    return p0 + p1


def _decoder_mm_body(nx_a, nx_b, o_ref):
    a = nx_a[...]
    b = nx_b[...]
    o_ref[...] = jax.lax.dot_general(
        a, b, (((1,), (1,)), ((), ())), preferred_element_type=jnp.float32)


def _decoder_matmul(node_x):
    TM = 1024
    grid = (pl.cdiv(N, TM), pl.cdiv(N, TM))
    return pl.pallas_call(
        _decoder_mm_body,
        grid=grid,
        in_specs=[
            pl.BlockSpec((TM, H), lambda i, j: (i, 0)),
            pl.BlockSpec((TM, H), lambda i, j: (j, 0)),
        ],
        out_specs=pl.BlockSpec((TM, TM), lambda i, j: (i, j)),
        out_shape=jax.ShapeDtypeStruct((N, N), jnp.float32),
    )(node_x, node_x)


def _gcn_conv_sc(x, src2, dst, W, b, dis):
    xw = x @ W
    xwp = xw * dis[:, None]
    scat = _sc_gcn_scatter(xwp, src2, dst)
    return dis[:, None] * (scat + xwp) + b


def _gatv2_conv(x, src2, dst2, dst, edge_attr, Wl, bl, Wr, br, We, att, b, n):
    xl = x @ Wl + bl
    xr = x @ Wr + br
    xl2 = jnp.concatenate([xl[:, :128], xl[:, 128:]], axis=0)
    xr2 = jnp.concatenate([xr[:, :128], xr[:, 128:]], axis=0)
    Gh = _gat_pass1(xl2, xr2, src2, dst2)
    G = jnp.concatenate([Gh[0], Gh[1]], axis=1)

    We_row = We[0]
    ea = edge_attr[:, 0]
    mean_ea = jnp.mean(ea)
    logit = jax.nn.leaky_relu(G + ea[:, None] * We_row, 0.2) @ att
    logit_self = jax.nn.leaky_relu(xl + xr + mean_ea * We_row, 0.2) @ att
    c0 = jnp.maximum(jnp.max(logit), jnp.max(logit_self))
    expl = jnp.exp(logit - c0)
    expl_self = jnp.exp(logit_self - c0)

    zeros1 = jnp.zeros((N,), jnp.float32)
    p0, p1 = _val_segsum(expl, dst, zeros1)
    den = p0 + p1 + expl_self

    zeros = jnp.zeros((N, 128), jnp.float32)
    expl16 = jnp.broadcast_to(expl[:, None], (E, 16)).reshape(E * 16)
    num = _colsplit_scatter_w(xl2, src2, dst, expl16, zeros)
    out = (num + expl_self[:, None] * xl) / den[:, None]
    return out + b


def kernel(x, edge_index, edge_attr, batch, gcn_W, gcn_b, gat_Wl, gat_bl,
           gat_Wr, gat_br, gat_We, gat_att, gat_b, gcn2_W, gcn2_b, lin1_W, lin1_b):
    src = edge_index[0]
    dst = edge_index[1]
    n = x.shape[0]
    src2 = jnp.concatenate([src, src + N])
    dst2 = jnp.concatenate([dst, dst + N])

    deg_part = _deg_kernel(dst, jnp.zeros((N,), jnp.float32))
    deg = deg_part[0] + deg_part[1] + 1.0
    dis = lax.rsqrt(deg)

    h = jax.nn.relu(_gcn_conv_sc(x, src2, dst, gcn_W, gcn_b, dis))
    h = jax.nn.relu(_gatv2_conv(h, src2, dst2, dst, edge_attr, gat_Wl, gat_bl,
                                gat_Wr, gat_br, gat_We, gat_att, gat_b, n))
    z = _gcn_conv_sc(h, src2, dst, gcn2_W, gcn2_b, dis)
    node_x = jax.nn.relu(z @ lin1_W + lin1_b)
    return _decoder_matmul(node_x)
